# Initial kernel scaffold; baseline (speedup 1.0000x reference)
import jax
import jax.numpy as jnp
from jax.experimental import pallas as pl


def kernel(x, edge_index, edge_attr, W_lin, b_lin, W_et, b_et, W_ih, b_ih, W_hh, b_hh, W_cls, b_cls):
    raise NotImplementedError("write your pallas kernel here")


# speedup vs baseline: 7.8457x; 7.8457x over previous
import functools

import jax
import jax.numpy as jnp
from jax import lax
from jax.experimental import pallas as pl
from jax.experimental.pallas import tpu as pltpu
from jax.experimental.pallas import tpu_sc as plsc

_N = 10000
_E = 320000
_D = 128
_T = 4
_STEPS = 8

_NC = 2
_NS = 16
_NW = _NC * _NS
_CHUNK = 128
_NCH = 79
_EPW = _NCH * _CHUNK
_EPAD = _NW * _EPW
_ASH_ROWS = 10240
_TRASH = _N

_NB = 2000
_GRID = _N // _NB


def _init_body(x_ref, wlin_ref, blin_ref, wet_ref, bet_ref, h_ref, p_ref):
    h = jnp.dot(x_ref[...], wlin_ref[...], preferred_element_type=jnp.float32)
    h = h + blin_ref[...]
    h_ref[...] = h
    for e in range(_T):
        p_ref[e] = (
            jnp.dot(h, wet_ref[e], preferred_element_type=jnp.float32)
            + bet_ref[e]
        )


def _tc_init(x, wlin_t, blin, wet_t, bet):
    return pl.pallas_call(
        _init_body,
        grid=(_GRID,),
        in_specs=[
            pl.BlockSpec((_NB, _D), lambda i: (i, 0)),
            pl.BlockSpec((_D, _D), lambda i: (0, 0)),
            pl.BlockSpec((1, _D), lambda i: (0, 0)),
            pl.BlockSpec((_T, _D, _D), lambda i: (0, 0, 0)),
            pl.BlockSpec((_T, 1, _D), lambda i: (0, 0, 0)),
        ],
        out_specs=[
            pl.BlockSpec((_NB, _D), lambda i: (i, 0)),
            pl.BlockSpec((_T, _NB, _D), lambda i: (0, i, 0)),
        ],
        out_shape=[
            jax.ShapeDtypeStruct((_N, _D), jnp.float32),
            jax.ShapeDtypeStruct((_T, _N, _D), jnp.float32),
        ],
    )(x, wlin_t, blin, wet_t, bet)


def _gru_body(ap_ref, h_ref, wih_ref, bih_ref, whh_ref, bhh_ref,
              wet_ref, bet_ref, hout_ref, pout_ref):
    a = ap_ref[0] + ap_ref[1]
    h = h_ref[...]
    gi = jnp.dot(a, wih_ref[...], preferred_element_type=jnp.float32)
    gi = gi + bih_ref[...]
    gh = jnp.dot(h, whh_ref[...], preferred_element_type=jnp.float32)
    gh = gh + bhh_ref[...]
    r = jax.nn.sigmoid(gi[:, 0:_D] + gh[:, 0:_D])
    z = jax.nn.sigmoid(gi[:, _D:2 * _D] + gh[:, _D:2 * _D])
    n = jnp.tanh(gi[:, 2 * _D:3 * _D] + r * gh[:, 2 * _D:3 * _D])
    hn = (1.0 - z) * n + z * h
    hout_ref[...] = hn
    for e in range(_T):
        pout_ref[e] = (
            jnp.dot(hn, wet_ref[e], preferred_element_type=jnp.float32)
            + bet_ref[e]
        )


def _tc_gru(apart, h, wih_t, bih, whh_t, bhh, wet_t, bet):
    return pl.pallas_call(
        _gru_body,
        grid=(_GRID,),
        in_specs=[
            pl.BlockSpec((_NC, _NB, _D), lambda i: (0, i, 0)),
            pl.BlockSpec((_NB, _D), lambda i: (i, 0)),
            pl.BlockSpec((_D, 3 * _D), lambda i: (0, 0)),
            pl.BlockSpec((1, 3 * _D), lambda i: (0, 0)),
            pl.BlockSpec((_D, 3 * _D), lambda i: (0, 0)),
            pl.BlockSpec((1, 3 * _D), lambda i: (0, 0)),
            pl.BlockSpec((_T, _D, _D), lambda i: (0, 0, 0)),
            pl.BlockSpec((_T, 1, _D), lambda i: (0, 0, 0)),
        ],
        out_specs=[
            pl.BlockSpec((_NB, _D), lambda i: (i, 0)),
            pl.BlockSpec((_T, _NB, _D), lambda i: (0, i, 0)),
        ],
        out_shape=[
            jax.ShapeDtypeStruct((_N, _D), jnp.float32),
            jax.ShapeDtypeStruct((_T, _N, _D), jnp.float32),
        ],
    )(apart, h, wih_t, bih, whh_t, bhh, wet_t, bet)


def _readout_body(h_ref, wcls_ref, bcls_ref, out_ref):
    i = pl.program_id(0)
    s = jnp.sum(h_ref[...], axis=0, keepdims=True)
    part = jnp.dot(s, wcls_ref[...], preferred_element_type=jnp.float32)

    @pl.when(i == 0)
    def _():
        out_ref[...] = bcls_ref[...]

    out_ref[...] += part * (1.0 / _N)


def _tc_readout(h, wcls_t, bcls):
    return pl.pallas_call(
        _readout_body,
        grid=(_GRID,),
        in_specs=[
            pl.BlockSpec((_NB, _D), lambda i: (i, 0)),
            pl.BlockSpec((_D, 2), lambda i: (0, 0)),
            pl.BlockSpec((1, 2), lambda i: (0, 0)),
        ],
        out_specs=pl.BlockSpec((1, 2), lambda i: (0, 0)),
        out_shape=jax.ShapeDtypeStruct((1, 2), jnp.float32),
    )(h, wcls_t, bcls)


@functools.partial(
    pl.kernel,
    out_type=jax.ShapeDtypeStruct((_NC, _ASH_ROWS, _D), jnp.float32),
    mesh=plsc.VectorSubcoreMesh(core_axis_name="c", subcore_axis_name="s"),
    scratch_types=[
        pltpu.VMEM((_CHUNK,), jnp.int32),
        pltpu.VMEM((_CHUNK,), jnp.int32),
        pltpu.VMEM((_CHUNK, _D), jnp.float32),
        pltpu.VMEM((16, _D), jnp.float32),
        pltpu.VMEM_SHARED((_ASH_ROWS, _D), jnp.float32),
        pltpu.SemaphoreType.DMA,
    ],
)
def _sc_scatter(p_hbm, gidx_hbm, dst_hbm, out_hbm,
                idx_v, dst_v, rows_v, zbuf, a_sh, sem):
    c = lax.axis_index("c")
    s = lax.axis_index("s")
    wid = c * _NS + s

    for r in range(16):
        for q in range(8):
            zbuf[r, pl.ds(q * 16, 16)] = jnp.zeros((16,), jnp.float32)

    def zero_step(j, carry):
        pltpu.sync_copy(zbuf, a_sh.at[pl.ds(s * 640 + j * 16, 16)])
        return carry

    lax.fori_loop(0, _ASH_ROWS // _NS // 16, zero_step, 0)
    plsc.subcore_barrier()

    base = wid * _EPW

    def chunk_step(j, carry):
        off = base + j * _CHUNK
        pltpu.sync_copy(gidx_hbm.at[pl.ds(off, _CHUNK)], idx_v)
        pltpu.sync_copy(dst_hbm.at[pl.ds(off, _CHUNK)], dst_v)
        pltpu.async_copy(p_hbm.at[idx_v], rows_v, sem).wait()
        pltpu.sync_copy(rows_v, a_sh.at[dst_v], add=True)
        return carry

    lax.fori_loop(0, _NCH, chunk_step, 0)
    plsc.subcore_barrier()

    rows_per = _ASH_ROWS // _NS
    pltpu.sync_copy(
        a_sh.at[pl.ds(s * rows_per, rows_per)],
        out_hbm.at[c, pl.ds(s * rows_per, rows_per)],
    )


def kernel(x, edge_index, edge_attr, W_lin, b_lin, W_et, b_et,
           W_ih, b_ih, W_hh, b_hh, W_cls, b_cls):
    src = edge_index[0]
    dst = edge_index[1]
    gidx = edge_attr * _N + src

    pad = _EPAD - _E
    gidx_p = jnp.concatenate([gidx, jnp.zeros((pad,), jnp.int32)])
    dst_p = jnp.concatenate([dst, jnp.full((pad,), _TRASH, jnp.int32)])

    wlin_t = W_lin.T
    blin = b_lin.reshape(1, _D)
    wet_t = jnp.transpose(W_et, (0, 2, 1))
    bet = b_et.reshape(_T, 1, _D)
    wih_t = W_ih.T
    bih = b_ih.reshape(1, 3 * _D)
    whh_t = W_hh.T
    bhh = b_hh.reshape(1, 3 * _D)
    wcls_t = W_cls.T
    bcls = b_cls.reshape(1, 2)

    h, p = _tc_init(x, wlin_t, blin, wet_t, bet)
    for _ in range(_STEPS):
        apart = _sc_scatter(p.reshape(_T * _N, _D), gidx_p, dst_p)
        h, p = _tc_gru(apart, h, wih_t, bih, whh_t, bhh, wet_t, bet)
    return _tc_readout(h, wcls_t, bcls)
